# Initial kernel scaffold; baseline (speedup 1.0000x reference)
import functools

import jax
import jax.numpy as jnp
from jax import lax
from jax.experimental import pallas as pl
from jax.experimental.pallas import tpu as pltpu
from jax.experimental.pallas import tpu_sc as plsc

_D = 64
_NC = 2
_NS = 16
_NW = _NC * _NS
_IDX_PER_STREAM = 128
_KG = 8
_CHUNK = _IDX_PER_STREAM * _KG


def _make_gather(n_rows: int):
    assert n_rows % (_NW * _CHUNK) == 0
    rows_per_w = n_rows // _NW
    n_groups = rows_per_w // _CHUNK
    mesh = plsc.VectorSubcoreMesh(core_axis_name="c", subcore_axis_name="s")

    @functools.partial(
        pl.kernel,
        out_type=jax.ShapeDtypeStruct((n_rows, _D), jnp.float32),
        mesh=mesh,
        scratch_types=[
            pltpu.VMEM((_CHUNK,), jnp.int32),
            pltpu.VMEM((_CHUNK, _D), jnp.float32),
            pltpu.SemaphoreType.DMA,
        ],
    )
    def gather(table_hbm, idx_hbm, out_hbm, idx_v, rows_v, sem):
        wid = lax.axis_index("s") * _NC + lax.axis_index("c")
        base = wid * rows_per_w

        def group(g, carry):
            off = base + g * _CHUNK
            pltpu.sync_copy(idx_hbm.at[pl.ds(off, _CHUNK)], idx_v)
            copies = []
            for j in range(_KG):
                s = j * _IDX_PER_STREAM
                copies.append(
                    pltpu.async_copy(
                        table_hbm.at[idx_v.at[pl.ds(s, _IDX_PER_STREAM)]],
                        rows_v.at[pl.ds(s, _IDX_PER_STREAM)],
                        sem,
                    )
                )
            for c in copies:
                c.wait()
            pltpu.sync_copy(rows_v, out_hbm.at[pl.ds(off, _CHUNK)])
            return carry

        lax.fori_loop(0, n_groups, group, 0)

    return gather


def kernel(actions, table):
    b, s = actions.shape
    idx = actions.reshape(-1).astype(jnp.int32)
    out = _make_gather(idx.shape[0])(table, idx)
    return out.reshape(b, s, _D)


# speedup vs baseline: 6.0585x; 6.0585x over previous
import functools

import jax
import jax.numpy as jnp
from jax import lax
from jax.experimental import pallas as pl
from jax.experimental.pallas import tpu as pltpu
from jax.experimental.pallas import tpu_sc as plsc

_D = 64
_NC = 2
_NS = 16
_NW = _NC * _NS
_IDX_PER_STREAM = 128
_KG = 8
_CHUNK = _IDX_PER_STREAM * _KG


def _make_gather(n_rows: int):
    assert n_rows % (_NW * _CHUNK) == 0
    rows_per_w = n_rows // _NW
    n_groups = rows_per_w // _CHUNK
    mesh = plsc.VectorSubcoreMesh(core_axis_name="c", subcore_axis_name="s")

    @functools.partial(
        pl.kernel,
        out_type=jax.ShapeDtypeStruct((n_rows, _D), jnp.float32),
        mesh=mesh,
        scratch_types=[
            pltpu.VMEM((_CHUNK,), jnp.int32),
            pltpu.VMEM((_CHUNK, _D), jnp.float32),
            pltpu.SemaphoreType.DMA,
        ],
        compiler_params=pltpu.CompilerParams(use_tc_tiling_on_sc=False),
    )
    def gather(table_hbm, idx_hbm, out_hbm, idx_v, rows_v, sem):
        wid = lax.axis_index("s") * _NC + lax.axis_index("c")
        base = wid * rows_per_w

        def group(g, carry):
            off = base + g * _CHUNK
            pltpu.sync_copy(idx_hbm.at[pl.ds(off, _CHUNK)], idx_v)
            copies = []
            for j in range(_KG):
                s = j * _IDX_PER_STREAM
                copies.append(
                    pltpu.async_copy(
                        table_hbm.at[idx_v.at[pl.ds(s, _IDX_PER_STREAM)]],
                        rows_v.at[pl.ds(s, _IDX_PER_STREAM)],
                        sem,
                    )
                )
            for c in copies:
                c.wait()
            pltpu.sync_copy(rows_v, out_hbm.at[pl.ds(off, _CHUNK)])
            return carry

        lax.fori_loop(0, n_groups, group, 0)

    return gather


def kernel(actions, table):
    b, s = actions.shape
    idx = actions.reshape(-1).astype(jnp.int32)
    out = _make_gather(idx.shape[0])(table, idx)
    return out.reshape(b, s, _D)


# speedup vs baseline: 6.2104x; 1.0251x over previous
import functools

import jax
import jax.numpy as jnp
from jax import lax
from jax.experimental import pallas as pl
from jax.experimental.pallas import tpu as pltpu
from jax.experimental.pallas import tpu_sc as plsc

_D = 64
_NC = 2
_NS = 16
_NW = _NC * _NS
_IDX_PER_STREAM = 128
_KG = 5
_CHUNK = _IDX_PER_STREAM * _KG


def _make_gather(n_rows: int):
    assert n_rows % (_NW * _CHUNK) == 0
    rows_per_w = n_rows // _NW
    n_groups = rows_per_w // _CHUNK
    assert n_groups % 2 == 0 and n_groups >= 4
    mesh = plsc.VectorSubcoreMesh(core_axis_name="c", subcore_axis_name="s")

    @functools.partial(
        pl.kernel,
        out_type=jax.ShapeDtypeStruct((n_rows, _D), jnp.float32),
        mesh=mesh,
        scratch_types=[
            pltpu.VMEM((rows_per_w,), jnp.int32),
            pltpu.VMEM((2, _CHUNK, _D), jnp.float32),
            pltpu.SemaphoreType.DMA,
            pltpu.SemaphoreType.DMA,
            pltpu.SemaphoreType.DMA,
        ],
        compiler_params=pltpu.CompilerParams(use_tc_tiling_on_sc=False),
    )
    def gather(table_hbm, idx_hbm, out_hbm, idx_v, rows_v, sem_i, sem_g, sem_w):
        wid = lax.axis_index("s") * _NC + lax.axis_index("c")
        base = wid * rows_per_w

        def fire_gathers(g, b):
            for j in range(_KG):
                s = j * _IDX_PER_STREAM
                pltpu.async_copy(
                    table_hbm.at[idx_v.at[pl.ds(g * _CHUNK + s, _IDX_PER_STREAM)]],
                    rows_v.at[b, pl.ds(s, _IDX_PER_STREAM)],
                    sem_g,
                )

        def drain_gathers(b):
            pltpu.make_async_copy(
                table_hbm.at[pl.ds(0, _CHUNK)], rows_v.at[b], sem_g
            ).wait()

        def fire_write(g, b):
            pltpu.async_copy(
                rows_v.at[b], out_hbm.at[pl.ds(base + g * _CHUNK, _CHUNK)], sem_w
            )

        def drain_write(b):
            pltpu.make_async_copy(
                table_hbm.at[pl.ds(0, _CHUNK)], rows_v.at[b], sem_w
            ).wait()

        pltpu.async_copy(idx_hbm.at[pl.ds(base, rows_per_w)], idx_v, sem_i).wait()

        fire_gathers(0, 0)
        drain_gathers(0)
        fire_write(0, 0)
        fire_gathers(1, 1)

        @pl.loop(0, n_groups - 2, step=2)
        def steady(k):
            for u in range(2):
                g = k + 1 + u
                b = (u + 1) % 2
                nb = 1 - b
                drain_gathers(b)
                fire_write(g, b)
                drain_write(nb)
                fire_gathers(g + 1, nb)

        drain_gathers(1)
        fire_write(n_groups - 1, 1)
        drain_write(0)
        drain_write(1)

    return gather


def kernel(actions, table):
    b, s = actions.shape
    idx = actions.reshape(-1).astype(jnp.int32)
    out = _make_gather(idx.shape[0])(table, idx)
    return out.reshape(b, s, _D)
